# Initial kernel scaffold; baseline (speedup 1.0000x reference)
import jax
import jax.numpy as jnp
from jax.experimental import pallas as pl


def kernel(inputs, targets):
    raise NotImplementedError("write your pallas kernel here")


# speedup vs baseline: 7.0629x; 7.0629x over previous
import functools

import jax
import jax.numpy as jnp
from jax import lax
from jax.experimental import pallas as pl
from jax.experimental.pallas import tpu as pltpu
from jax.experimental.pallas import tpu_sc as plsc

C = 19
L = 16
NC, NS = 2, 16
NW = NC * NS
B = 8
HW = 512 * 512
NPIX = B * HW
PIX_PER_TILE = NPIX // NW
TILES_PER_BATCH = HW // PIX_PER_TILE
CHUNK = 2048
NSUB = PIX_PER_TILE // CHUNK
NVEC = CHUNK // L
NBINS = 3 * C


@functools.partial(
    pl.kernel,
    out_type=(
        jax.ShapeDtypeStruct((C, NW * L), jnp.int32),
        jax.ShapeDtypeStruct((C, NW * L), jnp.int32),
        jax.ShapeDtypeStruct((C, NW * L), jnp.int32),
    ),
    mesh=plsc.VectorSubcoreMesh(core_axis_name="c", subcore_axis_name="s"),
    scratch_types=[
        pltpu.VMEM((C, CHUNK), jnp.float32),
        pltpu.VMEM((CHUNK,), jnp.int32),
        pltpu.VMEM((NBINS, L), jnp.int32),
    ],
    compiler_params=pltpu.CompilerParams(use_tc_tiling_on_sc=False,
                                         needs_layout_passes=False),
)
def _count_kernel(logits_hbm, targets_hbm, out_lo, out_hi, out_tg,
                  logit_v, tgt_v, hist_v):
    wid = lax.axis_index("s") * NC + lax.axis_index("c")
    batch = wid // TILES_PER_BATCH
    col_base = (wid % TILES_PER_BATCH) * PIX_PER_TILE
    pix_base = wid * PIX_PER_TILE

    zeros = jnp.zeros((L,), jnp.int32)
    for r in range(NBINS):
        hist_v[r, :] = zeros

    lane = lax.iota(jnp.int32, L)
    ones = jnp.ones((L,), jnp.int32)

    def chunk_body(s, carry):
        col0 = col_base + s * CHUNK
        pltpu.sync_copy(
            logits_hbm.at[pl.ds(batch * C, C), pl.ds(col0, CHUNK)], logit_v)
        pltpu.sync_copy(
            targets_hbm.at[pl.ds(pix_base + s * CHUNK, CHUNK)], tgt_v)

        def vec_body(i, carry2):
            off = i * L
            m = logit_v[0, pl.ds(off, L)]
            pred = jnp.zeros((L,), jnp.int32)
            for c in range(1, C):
                v = logit_v[c, pl.ds(off, L)]
                gt = v > m
                m = jnp.where(gt, v, m)
                pred = jnp.where(gt, jnp.full((L,), c, jnp.int32), pred)
            t = tgt_v[pl.ds(off, L)]
            key = jnp.where(pred == t, pred + C, pred)
            plsc.addupdate_scatter(hist_v, [key, lane], ones)
            plsc.addupdate_scatter(hist_v, [t + 2 * C, lane], ones)
            return carry2

        return lax.fori_loop(0, NVEC, vec_body, carry)

    lax.fori_loop(0, NSUB, chunk_body, 0)

    col = pl.ds(wid * L, L)
    pltpu.sync_copy(hist_v.at[pl.ds(0, C)], out_lo.at[:, col])
    pltpu.sync_copy(hist_v.at[pl.ds(C, C)], out_hi.at[:, col])
    pltpu.sync_copy(hist_v.at[pl.ds(2 * C, C)], out_tg.at[:, col])


def _combine_body(lo_ref, hi_ref, tg_ref, out_ref):
    lo = jnp.sum(lo_ref[...].astype(jnp.float32), axis=1, keepdims=True)
    hi = jnp.sum(hi_ref[...].astype(jnp.float32), axis=1, keepdims=True)
    tg = jnp.sum(tg_ref[...].astype(jnp.float32), axis=1, keepdims=True)
    inter = hi
    pred = lo + hi
    union = pred + tg - inter
    iou = inter / (union + 1e-16)
    out_ref[0, 0] = jnp.sum(iou) / float(C)


_combine = pl.pallas_call(
    _combine_body,
    out_shape=jax.ShapeDtypeStruct((1, 1), jnp.float32),
    out_specs=pl.BlockSpec(memory_space=pltpu.SMEM),
)


def kernel(inputs, targets):
    logits = inputs.reshape(B * C, HW)
    tgt = targets.reshape(NPIX)
    lo, hi, tg = _count_kernel(logits, tgt)
    return _combine(lo, hi, tg)[0, 0]


# speedup vs baseline: 9.2128x; 1.3044x over previous
import functools

import jax
import jax.numpy as jnp
from jax import lax
from jax.experimental import pallas as pl
from jax.experimental.pallas import tpu as pltpu
from jax.experimental.pallas import tpu_sc as plsc

C = 19
L = 16
NC, NS = 2, 16
NW = NC * NS
B = 8
HW = 512 * 512
NPIX = B * HW
PIX_PER_TILE = NPIX // NW
TILES_PER_BATCH = HW // PIX_PER_TILE
CHUNK = 2048
NSUB = PIX_PER_TILE // CHUNK
NVEC = CHUNK // L
UNROLL = 4
NBINS = 3 * C


@functools.partial(
    pl.kernel,
    out_type=(
        jax.ShapeDtypeStruct((C, NW * L), jnp.int32),
        jax.ShapeDtypeStruct((C, NW * L), jnp.int32),
        jax.ShapeDtypeStruct((C, NW * L), jnp.int32),
    ),
    mesh=plsc.VectorSubcoreMesh(core_axis_name="c", subcore_axis_name="s"),
    scratch_types=[
        pltpu.VMEM((2, C, CHUNK), jnp.float32),
        pltpu.VMEM((2, CHUNK), jnp.int32),
        pltpu.VMEM((NBINS, L), jnp.int32),
        pltpu.SemaphoreType.DMA,
        pltpu.SemaphoreType.DMA,
    ],
    compiler_params=pltpu.CompilerParams(use_tc_tiling_on_sc=False,
                                         needs_layout_passes=False),
)
def _count_kernel(logits_hbm, targets_hbm, out_lo, out_hi, out_tg,
                  logit_v, tgt_v, hist_v, sem0, sem1):
    wid = lax.axis_index("s") * NC + lax.axis_index("c")
    batch = wid // TILES_PER_BATCH
    col_base = (wid % TILES_PER_BATCH) * PIX_PER_TILE
    pix_base = wid * PIX_PER_TILE
    row0 = batch * C

    zeros = jnp.zeros((L,), jnp.int32)
    for r in range(NBINS):
        hist_v[r, :] = zeros

    lane = lax.iota(jnp.int32, L)
    ones = jnp.ones((L,), jnp.int32)
    sems = (sem0, sem1)

    def copies(chunk, buf, sem):
        col0 = col_base + chunk * CHUNK
        return (
            pltpu.make_async_copy(
                logits_hbm.at[pl.ds(row0, C), pl.ds(col0, CHUNK)],
                logit_v.at[buf], sem),
            pltpu.make_async_copy(
                targets_hbm.at[pl.ds(pix_base + chunk * CHUNK, CHUNK)],
                tgt_v.at[buf], sem),
        )

    def start(chunk, buf, sem):
        for cp in copies(chunk, buf, sem):
            cp.start()

    def wait(chunk, buf, sem):
        for cp in copies(chunk, buf, sem):
            cp.wait()

    start(0, 0, sem0)
    start(1, 1, sem1)

    def compute_chunk(buf):
        def vec_body(j, carry2):
            for u in range(UNROLL):
                off = (j * UNROLL + u) * L
                m = logit_v[buf, 0, pl.ds(off, L)]
                pred = jnp.zeros((L,), jnp.int32)
                for c in range(1, C):
                    v = logit_v[buf, c, pl.ds(off, L)]
                    gt = v > m
                    m = jnp.where(gt, v, m)
                    pred = jnp.where(gt, jnp.full((L,), c, jnp.int32), pred)
                t = tgt_v[buf, pl.ds(off, L)]
                key = jnp.where(pred == t, pred + C, pred)
                plsc.addupdate_scatter(hist_v, [key, lane], ones)
                plsc.addupdate_scatter(hist_v, [t + 2 * C, lane], ones)
            return carry2

        lax.fori_loop(0, NVEC // UNROLL, vec_body, 0)

    def pair_body(i, carry):
        base = 2 * i
        for p in range(2):
            chunk = base + p
            wait(chunk, p, sems[p])
            compute_chunk(p)

            @pl.when(chunk + 2 < NSUB)
            def _():
                start(chunk + 2, p, sems[p])
        return carry

    lax.fori_loop(0, NSUB // 2, pair_body, 0)

    col = pl.ds(wid * L, L)
    pltpu.sync_copy(hist_v.at[pl.ds(0, C)], out_lo.at[:, col])
    pltpu.sync_copy(hist_v.at[pl.ds(C, C)], out_hi.at[:, col])
    pltpu.sync_copy(hist_v.at[pl.ds(2 * C, C)], out_tg.at[:, col])


def _combine_body(lo_ref, hi_ref, tg_ref, out_ref):
    lo = jnp.sum(lo_ref[...].astype(jnp.float32), axis=1, keepdims=True)
    hi = jnp.sum(hi_ref[...].astype(jnp.float32), axis=1, keepdims=True)
    tg = jnp.sum(tg_ref[...].astype(jnp.float32), axis=1, keepdims=True)
    inter = hi
    pred = lo + hi
    union = pred + tg - inter
    iou = inter / (union + 1e-16)
    out_ref[0, 0] = jnp.sum(iou) / float(C)


_combine = pl.pallas_call(
    _combine_body,
    out_shape=jax.ShapeDtypeStruct((1, 1), jnp.float32),
    out_specs=pl.BlockSpec(memory_space=pltpu.SMEM),
)


def kernel(inputs, targets):
    logits = inputs.reshape(B * C, HW)
    tgt = targets.reshape(NPIX)
    lo, hi, tg = _count_kernel(logits, tgt)
    return _combine(lo, hi, tg)[0, 0]


# speedup vs baseline: 16.7131x; 1.8141x over previous
import functools

import jax
import jax.numpy as jnp
from jax import lax
from jax.experimental import pallas as pl
from jax.experimental.pallas import tpu as pltpu
from jax.experimental.pallas import tpu_sc as plsc

C = 19
L = 16
NC, NS = 2, 16
NW = NC * NS
B = 8
H = 512
W = 512
ROWS = 8
COLS = 256
CHUNK_PX = ROWS * COLS
CHUNKS_PER_IMG = (H // ROWS) * (W // COLS)
NCHUNKS = B * CHUNKS_PER_IMG
CHUNKS_PER_TILE = NCHUNKS // NW
UNROLL = 4


@functools.partial(
    pl.kernel,
    out_type=(
        jax.ShapeDtypeStruct((NW, C, L), jnp.int32),
        jax.ShapeDtypeStruct((NW, C, L), jnp.int32),
        jax.ShapeDtypeStruct((NW, C, L), jnp.int32),
    ),
    mesh=plsc.VectorSubcoreMesh(core_axis_name="c", subcore_axis_name="s"),
    scratch_types=[
        pltpu.VMEM((2, C, ROWS, COLS), jnp.float32),
        pltpu.VMEM((2, ROWS, COLS), jnp.int32),
        pltpu.VMEM((C, L), jnp.int32),
        pltpu.VMEM((C, L), jnp.int32),
        pltpu.VMEM((C, L), jnp.int32),
        pltpu.SemaphoreType.DMA,
        pltpu.SemaphoreType.DMA,
    ],
    compiler_params=pltpu.CompilerParams(needs_layout_passes=False),
)
def _count_kernel(logits_hbm, targets_hbm, out_lo, out_hi, out_tg,
                  logit_v, tgt_v, hist_lo, hist_hi, hist_tg, sem0, sem1):
    wid = lax.axis_index("s") * NC + lax.axis_index("c")
    q_base = wid * CHUNKS_PER_TILE

    zeros = jnp.zeros((L,), jnp.int32)
    for r in range(C):
        hist_lo[r, :] = zeros
        hist_hi[r, :] = zeros
        hist_tg[r, :] = zeros

    lane = lax.iota(jnp.int32, L)
    ones = jnp.ones((L,), jnp.int32)
    sems = (sem0, sem1)

    def copies(q, buf, sem):
        b = q // CHUNKS_PER_IMG
        rem = q % CHUNKS_PER_IMG
        r0 = (rem // 2) * ROWS
        c0 = (rem % 2) * COLS
        return (
            pltpu.make_async_copy(
                logits_hbm.at[b, :, pl.ds(r0, ROWS), pl.ds(c0, COLS)],
                logit_v.at[buf], sem),
            pltpu.make_async_copy(
                targets_hbm.at[b, pl.ds(r0, ROWS), pl.ds(c0, COLS)],
                tgt_v.at[buf], sem),
        )

    def start(q, buf, sem):
        for cp in copies(q, buf, sem):
            cp.start()

    def wait(q, buf, sem):
        for cp in copies(q, buf, sem):
            cp.wait()

    start(q_base, 0, sem0)
    start(q_base + 1, 1, sem1)

    def compute_chunk(buf):
        def row_body(r, carry1):
            def vec_body(j, carry2):
                for u in range(UNROLL):
                    off = (j * UNROLL + u) * L
                    m = logit_v[buf, 0, r, pl.ds(off, L)]
                    pred = jnp.zeros((L,), jnp.int32)
                    for c in range(1, C):
                        v = logit_v[buf, c, r, pl.ds(off, L)]
                        gt = v > m
                        m = jnp.where(gt, v, m)
                        pred = jnp.where(gt, jnp.full((L,), c, jnp.int32),
                                         pred)
                    t = tgt_v[buf, r, pl.ds(off, L)]
                    eqm = pred == t
                    plsc.addupdate_scatter(hist_hi, [pred, lane], ones,
                                           mask=eqm)
                    plsc.addupdate_scatter(hist_lo, [pred, lane], ones,
                                           mask=jnp.logical_not(eqm))
                    plsc.addupdate_scatter(hist_tg, [t, lane], ones)
                return carry2

            return lax.fori_loop(0, COLS // (UNROLL * L), vec_body, carry1)

        lax.fori_loop(0, ROWS, row_body, 0)

    def pair_body(i, carry):
        base = q_base + 2 * i
        for p in range(2):
            q = base + p
            wait(q, p, sems[p])
            compute_chunk(p)

            @pl.when(2 * i + p + 2 < CHUNKS_PER_TILE)
            def _():
                start(q + 2, p, sems[p])
        return carry

    lax.fori_loop(0, CHUNKS_PER_TILE // 2, pair_body, 0)

    pltpu.sync_copy(hist_lo, out_lo.at[wid])
    pltpu.sync_copy(hist_hi, out_hi.at[wid])
    pltpu.sync_copy(hist_tg, out_tg.at[wid])


def _combine_body(lo_ref, hi_ref, tg_ref, out_ref):
    lo = jnp.sum(lo_ref[...].astype(jnp.float32), axis=(0, 2), keepdims=True)
    hi = jnp.sum(hi_ref[...].astype(jnp.float32), axis=(0, 2), keepdims=True)
    tg = jnp.sum(tg_ref[...].astype(jnp.float32), axis=(0, 2), keepdims=True)
    inter = hi
    pred = lo + hi
    union = pred + tg - inter
    iou = inter / (union + 1e-16)
    out_ref[0, 0] = jnp.sum(iou) / float(C)


_combine = pl.pallas_call(
    _combine_body,
    out_shape=jax.ShapeDtypeStruct((1, 1), jnp.float32),
    out_specs=pl.BlockSpec(memory_space=pltpu.SMEM),
)


def kernel(inputs, targets):
    lo, hi, tg = _count_kernel(inputs, targets)
    return _combine(lo, hi, tg)[0, 0]


# speedup vs baseline: 20.2450x; 1.2113x over previous
import functools

import jax
import jax.numpy as jnp
from jax import lax
from jax.experimental import pallas as pl
from jax.experimental.pallas import tpu as pltpu
from jax.experimental.pallas import tpu_sc as plsc

C = 19
L = 16
NC, NS = 2, 16
NW = NC * NS
B = 8
H = 512
W = 512
ROWS = 8
COLS = 256
CHUNK_PX = ROWS * COLS
CHUNKS_PER_IMG = (H // ROWS) * (W // COLS)
NCHUNKS = B * CHUNKS_PER_IMG
CHUNKS_PER_TILE = NCHUNKS // NW
UNROLL = 2


def _argmax_tree(vals):
    n = len(vals)
    nodes = []
    for a in range(0, n - 1, 2):
        gt = vals[a + 1] > vals[a]
        v = jnp.where(gt, vals[a + 1], vals[a])
        i = jnp.where(gt, jnp.full((L,), a + 1, jnp.int32),
                      jnp.full((L,), a, jnp.int32))
        nodes.append((v, i))
    if n % 2:
        nodes.append((vals[n - 1], jnp.full((L,), n - 1, jnp.int32)))
    while len(nodes) > 1:
        nxt = []
        for k in range(0, len(nodes) - 1, 2):
            (va, ia), (vb, ib) = nodes[k], nodes[k + 1]
            gt = vb > va
            nxt.append((jnp.where(gt, vb, va), jnp.where(gt, ib, ia)))
        if len(nodes) % 2:
            nxt.append(nodes[-1])
        nodes = nxt
    return nodes[0][1]


@functools.partial(
    pl.kernel,
    out_type=(
        jax.ShapeDtypeStruct((NW, C, L), jnp.int32),
        jax.ShapeDtypeStruct((NW, C, L), jnp.int32),
        jax.ShapeDtypeStruct((NW, C, L), jnp.int32),
    ),
    mesh=plsc.VectorSubcoreMesh(core_axis_name="c", subcore_axis_name="s"),
    scratch_types=[
        pltpu.VMEM((2, C, ROWS, COLS), jnp.float32),
        pltpu.VMEM((2, ROWS, COLS), jnp.int32),
        pltpu.VMEM((C, L), jnp.int32),
        pltpu.VMEM((C, L), jnp.int32),
        pltpu.VMEM((C, L), jnp.int32),
        pltpu.SemaphoreType.DMA,
        pltpu.SemaphoreType.DMA,
    ],
    compiler_params=pltpu.CompilerParams(needs_layout_passes=False),
)
def _count_kernel(logits_hbm, targets_hbm, out_lo, out_hi, out_tg,
                  logit_v, tgt_v, hist_lo, hist_hi, hist_tg, sem0, sem1):
    wid = lax.axis_index("s") * NC + lax.axis_index("c")
    q_base = wid * CHUNKS_PER_TILE

    zeros = jnp.zeros((L,), jnp.int32)
    for r in range(C):
        hist_lo[r, :] = zeros
        hist_hi[r, :] = zeros
        hist_tg[r, :] = zeros

    lane = lax.iota(jnp.int32, L)
    ones = jnp.ones((L,), jnp.int32)
    sems = (sem0, sem1)

    def copies(q, buf, sem):
        b = q // CHUNKS_PER_IMG
        rem = q % CHUNKS_PER_IMG
        r0 = (rem // 2) * ROWS
        c0 = (rem % 2) * COLS
        return (
            pltpu.make_async_copy(
                logits_hbm.at[b, :, pl.ds(r0, ROWS), pl.ds(c0, COLS)],
                logit_v.at[buf], sem),
            pltpu.make_async_copy(
                targets_hbm.at[b, pl.ds(r0, ROWS), pl.ds(c0, COLS)],
                tgt_v.at[buf], sem),
        )

    def start(q, buf, sem):
        for cp in copies(q, buf, sem):
            cp.start()

    def wait(q, buf, sem):
        for cp in copies(q, buf, sem):
            cp.wait()

    start(q_base, 0, sem0)
    start(q_base + 1, 1, sem1)

    def compute_chunk(buf):
        def row_body(r, carry1):
            def vec_body(j, carry2):
                for u in range(UNROLL):
                    off = (j * UNROLL + u) * L
                    vals = [logit_v[buf, c, r, pl.ds(off, L)]
                            for c in range(C)]
                    pred = _argmax_tree(vals)
                    t = tgt_v[buf, r, pl.ds(off, L)]
                    eqm = pred == t
                    plsc.addupdate_scatter(hist_hi, [pred, lane], ones,
                                           mask=eqm)
                    plsc.addupdate_scatter(hist_lo, [pred, lane], ones,
                                           mask=jnp.logical_not(eqm))
                    plsc.addupdate_scatter(hist_tg, [t, lane], ones)
                return carry2

            return lax.fori_loop(0, COLS // (UNROLL * L), vec_body, carry1)

        lax.fori_loop(0, ROWS, row_body, 0)

    def pair_body(i, carry):
        base = q_base + 2 * i
        for p in range(2):
            q = base + p
            wait(q, p, sems[p])
            compute_chunk(p)

            @pl.when(2 * i + p + 2 < CHUNKS_PER_TILE)
            def _():
                start(q + 2, p, sems[p])
        return carry

    lax.fori_loop(0, CHUNKS_PER_TILE // 2, pair_body, 0)

    pltpu.sync_copy(hist_lo, out_lo.at[wid])
    pltpu.sync_copy(hist_hi, out_hi.at[wid])
    pltpu.sync_copy(hist_tg, out_tg.at[wid])


def _combine_body(lo_ref, hi_ref, tg_ref, out_ref):
    lo = jnp.sum(lo_ref[...].astype(jnp.float32), axis=(0, 2), keepdims=True)
    hi = jnp.sum(hi_ref[...].astype(jnp.float32), axis=(0, 2), keepdims=True)
    tg = jnp.sum(tg_ref[...].astype(jnp.float32), axis=(0, 2), keepdims=True)
    inter = hi
    pred = lo + hi
    union = pred + tg - inter
    iou = inter / (union + 1e-16)
    out_ref[0, 0] = jnp.sum(iou) / float(C)


_combine = pl.pallas_call(
    _combine_body,
    out_shape=jax.ShapeDtypeStruct((1, 1), jnp.float32),
    out_specs=pl.BlockSpec(memory_space=pltpu.SMEM),
)


def kernel(inputs, targets):
    lo, hi, tg = _count_kernel(inputs, targets)
    return _combine(lo, hi, tg)[0, 0]
